# baseline (device time: 194746 ns/iter reference)
import jax
import jax.numpy as jnp
from jax import lax
from jax.experimental import pallas as pl
from jax.experimental.pallas import tpu as pltpu

N_DEV = 8
B = 2
S = 256
HQ = 4
DH = 64
D_MODEL = 512
D_QK = 256
N_HOPS = N_DEV - 1


def kernel(x, Wq, K_ext, V_ext, Wo):
    def body(x_ref, wq_ref, k_ref, v_ref, wo_ref, out_ref,
             kcomm, vcomm, ksend, krecv, vsend, vrecv):
        my = lax.axis_index("i")
        left = (my - 1) % N_DEV
        right = (my + 1) % N_DEV

        barrier_sem = pltpu.get_barrier_semaphore()
        for nbr in (left, right):
            pl.semaphore_signal(
                barrier_sem, inc=1,
                device_id=(nbr,), device_id_type=pl.DeviceIdType.MESH,
            )
        pl.semaphore_wait(barrier_sem, 2)

        kcomm[0] = k_ref[...]
        vcomm[0] = v_ref[...]

        for h in range(N_HOPS):
            kdma = pltpu.make_async_remote_copy(
                src_ref=kcomm.at[h], dst_ref=kcomm.at[h + 1],
                send_sem=ksend.at[h], recv_sem=krecv.at[h],
                device_id=(right,), device_id_type=pl.DeviceIdType.MESH,
            )
            vdma = pltpu.make_async_remote_copy(
                src_ref=vcomm.at[h], dst_ref=vcomm.at[h + 1],
                send_sem=vsend.at[h], recv_sem=vrecv.at[h],
                device_id=(right,), device_id_type=pl.DeviceIdType.MESH,
            )
            kdma.start()
            vdma.start()
            kdma.wait()
            vdma.wait()

        rowres = (lax.broadcasted_iota(jnp.int32, (S, N_DEV * S), 0) // 64) % 4
        colres = (lax.broadcasted_iota(jnp.int32, (S, N_DEV * S), 1) // 64) % 4
        mask = rowres == colres

        for b in range(B):
            q = jnp.dot(x_ref[b], wq_ref[...],
                        preferred_element_type=jnp.float32)
            ctx_heads = []
            for h in range(HQ):
                qh = q[:, h * DH:(h + 1) * DH]
                scores = jnp.concatenate(
                    [
                        lax.dot_general(
                            qh, kcomm[s, b, :, h, :],
                            (((1,), (1,)), ((), ())),
                            preferred_element_type=jnp.float32,
                        )
                        for s in range(N_DEV)
                    ],
                    axis=1,
                ) * 0.125
                scores = jnp.where(mask, scores, -1e9)
                m = jnp.max(scores, axis=1, keepdims=True)
                w = jnp.exp(scores - m)
                w = w / jnp.sum(w, axis=1, keepdims=True)
                vcat = jnp.concatenate(
                    [vcomm[s, b, :, h, :] for s in range(N_DEV)], axis=0
                )
                ctx_heads.append(
                    jnp.dot(w, vcat, preferred_element_type=jnp.float32)
                )
            ctx = jnp.concatenate(ctx_heads, axis=1)
            out_ref[b] = jnp.dot(ctx, wo_ref[...],
                                 preferred_element_type=jnp.float32)

    return pl.pallas_call(
        body,
        out_shape=jax.ShapeDtypeStruct((B, S, D_MODEL), jnp.float32),
        in_specs=[pl.BlockSpec(memory_space=pltpu.VMEM)] * 5,
        out_specs=pl.BlockSpec(memory_space=pltpu.VMEM),
        scratch_shapes=[
            pltpu.VMEM((N_DEV, B, S, HQ, DH), jnp.float32),
            pltpu.VMEM((N_DEV, B, S, HQ, DH), jnp.float32),
            pltpu.SemaphoreType.DMA((N_HOPS,)),
            pltpu.SemaphoreType.DMA((N_HOPS,)),
            pltpu.SemaphoreType.DMA((N_HOPS,)),
            pltpu.SemaphoreType.DMA((N_HOPS,)),
        ],
        compiler_params=pltpu.CompilerParams(collective_id=0),
    )(x, Wq, K_ext, V_ext, Wo)


# device time: 111561 ns/iter; 1.7456x vs baseline; 1.7456x over previous
import os

import jax
import jax.numpy as jnp
from jax import lax
from jax.experimental import pallas as pl
from jax.experimental.pallas import tpu as pltpu

os.makedirs("/tmp/jax_kernel_cache", exist_ok=True)
jax.config.update("jax_compilation_cache_dir", "/tmp/jax_kernel_cache")
jax.config.update("jax_persistent_cache_min_compile_time_secs", 0)
jax.config.update("jax_persistent_cache_min_entry_size_bytes", -1)

N_DEV = 8
B = 2
S = 256
HQ = 4
DH = 64
D_MODEL = 512
D_QK = 256
NB = 4
BLK = 64


def kernel(x, Wq, K_ext, V_ext, Wo):
    def body(x_ref, wq_ref, k_ref, v_ref, wo_ref, out_ref,
             kcomm, vcomm, ksend, krecv, vsend, vrecv):
        my = lax.axis_index("i")
        px = my ^ 1
        py = jnp.where(my < 4, 3 - my, 11 - my)
        pz = my ^ 4

        barrier_sem = pltpu.get_barrier_semaphore()
        for p in (px, py, pz):
            pl.semaphore_signal(
                barrier_sem, inc=1,
                device_id=(p,), device_id_type=pl.DeviceIdType.MESH,
            )
        pl.semaphore_wait(barrier_sem, 3)

        kcomm[0] = k_ref[...].astype(jnp.bfloat16)
        vcomm[0] = v_ref[...].astype(jnp.bfloat16)

        def make(phase, cnt, partner):
            kdma = pltpu.make_async_remote_copy(
                src_ref=kcomm.at[pl.ds(0, cnt)],
                dst_ref=kcomm.at[pl.ds(cnt, cnt)],
                send_sem=ksend.at[phase], recv_sem=krecv.at[phase],
                device_id=(partner,), device_id_type=pl.DeviceIdType.MESH,
            )
            vdma = pltpu.make_async_remote_copy(
                src_ref=vcomm.at[pl.ds(0, cnt)],
                dst_ref=vcomm.at[pl.ds(cnt, cnt)],
                send_sem=vsend.at[phase], recv_sem=vrecv.at[phase],
                device_id=(partner,), device_id_type=pl.DeviceIdType.MESH,
            )
            return kdma, vdma

        kdma, vdma = make(0, 1, px)
        kdma.start()
        vdma.start()
        wq_bf = wq_ref[...].astype(jnp.bfloat16)
        qs = [
            jnp.dot(x_ref[b].astype(jnp.bfloat16), wq_bf,
                    preferred_element_type=jnp.float32).astype(jnp.bfloat16)
            for b in range(B)
        ]
        kdma.wait()
        vdma.wait()

        kdma, vdma = make(1, 2, py)
        kdma.start()
        vdma.start()
        kdma.wait()
        vdma.wait()

        kdma, vdma = make(2, 4, pz)
        kdma.start()
        vdma.start()
        kdma.wait()
        vdma.wait()

        for b in range(B):
            ctx_heads = []
            for h in range(HQ):
                qh = qs[b][:, h * DH:(h + 1) * DH].reshape(NB, BLK, DH)
                k4 = jnp.concatenate(
                    [kcomm[s, b, :, h, :].reshape(NB, BLK, DH)
                     for s in range(N_DEV)], axis=1)
                v4 = jnp.concatenate(
                    [vcomm[s, b, :, h, :].reshape(NB, BLK, DH)
                     for s in range(N_DEV)], axis=1)
                scores = lax.dot_general(
                    qh, k4, (((2,), (2,)), ((0,), (0,))),
                    preferred_element_type=jnp.float32,
                ) * 0.125
                m = jnp.max(scores, axis=2, keepdims=True)
                w = jnp.exp(scores - m)
                w = (w / jnp.sum(w, axis=2, keepdims=True)).astype(jnp.bfloat16)
                ctx_r = lax.dot_general(
                    w, v4, (((2,), (1,)), ((0,), (0,))),
                    preferred_element_type=jnp.float32,
                )
                ctx_heads.append(ctx_r.reshape(S, DH))
            ctx = jnp.concatenate(ctx_heads, axis=1).astype(jnp.bfloat16)
            out_ref[b] = jnp.dot(ctx, wo_ref[...].astype(jnp.bfloat16),
                                 preferred_element_type=jnp.float32)

    return pl.pallas_call(
        body,
        out_shape=jax.ShapeDtypeStruct((B, S, D_MODEL), jnp.float32),
        in_specs=[pl.BlockSpec(memory_space=pltpu.VMEM)] * 5,
        out_specs=pl.BlockSpec(memory_space=pltpu.VMEM),
        scratch_shapes=[
            pltpu.VMEM((N_DEV, B, S, HQ, DH), jnp.bfloat16),
            pltpu.VMEM((N_DEV, B, S, HQ, DH), jnp.bfloat16),
            pltpu.SemaphoreType.DMA((3,)),
            pltpu.SemaphoreType.DMA((3,)),
            pltpu.SemaphoreType.DMA((3,)),
            pltpu.SemaphoreType.DMA((3,)),
        ],
        compiler_params=pltpu.CompilerParams(collective_id=0),
    )(x, Wq, K_ext, V_ext, Wo)


# device time: 86646 ns/iter; 2.2476x vs baseline; 1.2875x over previous
import os

import jax
import jax.numpy as jnp
from jax import lax
from jax.experimental import pallas as pl
from jax.experimental.pallas import tpu as pltpu

os.makedirs("/tmp/jax_kernel_cache", exist_ok=True)
jax.config.update("jax_compilation_cache_dir", "/tmp/jax_kernel_cache")
jax.config.update("jax_persistent_cache_min_compile_time_secs", 0)
jax.config.update("jax_persistent_cache_min_entry_size_bytes", -1)

N_DEV = 8
B = 2
S = 256
HQ = 4
DH = 64
D_MODEL = 512
D_QK = 256
NB = 4
BLK = 64


def kernel(x, Wq, K_ext, V_ext, Wo):
    def body(x_ref, wq_ref, k_ref, v_ref, wo_ref, out_ref,
             kcomm, vcomm, ksend, krecv, vsend, vrecv):
        my = lax.axis_index("i")

        barrier_sem = pltpu.get_barrier_semaphore()
        for d in range(1, N_DEV):
            pl.semaphore_signal(
                barrier_sem, inc=1,
                device_id=(my ^ d,), device_id_type=pl.DeviceIdType.MESH,
            )
        pl.semaphore_wait(barrier_sem, N_DEV - 1)

        kcomm[0] = k_ref[...].astype(jnp.bfloat16)
        vcomm[0] = v_ref[...].astype(jnp.bfloat16)

        dmas = []
        for d in range(1, N_DEV):
            kdma = pltpu.make_async_remote_copy(
                src_ref=kcomm.at[0], dst_ref=kcomm.at[d],
                send_sem=ksend.at[d - 1], recv_sem=krecv.at[d - 1],
                device_id=(my ^ d,), device_id_type=pl.DeviceIdType.MESH,
            )
            vdma = pltpu.make_async_remote_copy(
                src_ref=vcomm.at[0], dst_ref=vcomm.at[d],
                send_sem=vsend.at[d - 1], recv_sem=vrecv.at[d - 1],
                device_id=(my ^ d,), device_id_type=pl.DeviceIdType.MESH,
            )
            kdma.start()
            vdma.start()
            dmas.append((kdma, vdma))

        wq_bf = wq_ref[...].astype(jnp.bfloat16)
        qs = [
            jnp.dot(x_ref[b].astype(jnp.bfloat16), wq_bf,
                    preferred_element_type=jnp.float32).astype(jnp.bfloat16)
            for b in range(B)
        ]

        for kdma, vdma in dmas:
            kdma.wait()
            vdma.wait()

        for b in range(B):
            ctx_heads = []
            for h in range(HQ):
                qh = qs[b][:, h * DH:(h + 1) * DH].reshape(NB, BLK, DH)
                k4 = jnp.concatenate(
                    [kcomm[s, b, :, h, :].reshape(NB, BLK, DH)
                     for s in range(N_DEV)], axis=1)
                v4 = jnp.concatenate(
                    [vcomm[s, b, :, h, :].reshape(NB, BLK, DH)
                     for s in range(N_DEV)], axis=1)
                scores = lax.dot_general(
                    qh, k4, (((2,), (2,)), ((0,), (0,))),
                    preferred_element_type=jnp.float32,
                ) * 0.125
                m = jnp.max(scores, axis=2, keepdims=True)
                w = jnp.exp(scores - m)
                w = (w / jnp.sum(w, axis=2, keepdims=True)).astype(jnp.bfloat16)
                ctx_r = lax.dot_general(
                    w, v4, (((2,), (1,)), ((0,), (0,))),
                    preferred_element_type=jnp.float32,
                )
                ctx_heads.append(ctx_r.reshape(S, DH))
            ctx = jnp.concatenate(ctx_heads, axis=1).astype(jnp.bfloat16)
            out_ref[b] = jnp.dot(ctx, wo_ref[...].astype(jnp.bfloat16),
                                 preferred_element_type=jnp.float32)

    return pl.pallas_call(
        body,
        out_shape=jax.ShapeDtypeStruct((B, S, D_MODEL), jnp.float32),
        in_specs=[pl.BlockSpec(memory_space=pltpu.VMEM)] * 5,
        out_specs=pl.BlockSpec(memory_space=pltpu.VMEM),
        scratch_shapes=[
            pltpu.VMEM((N_DEV, B, S, HQ, DH), jnp.bfloat16),
            pltpu.VMEM((N_DEV, B, S, HQ, DH), jnp.bfloat16),
            pltpu.SemaphoreType.DMA((N_DEV - 1,)),
            pltpu.SemaphoreType.DMA((N_DEV - 1,)),
            pltpu.SemaphoreType.DMA((N_DEV - 1,)),
            pltpu.SemaphoreType.DMA((N_DEV - 1,)),
        ],
        compiler_params=pltpu.CompilerParams(collective_id=0),
    )(x, Wq, K_ext, V_ext, Wo)


# device time: 23745 ns/iter; 8.2016x vs baseline; 3.6490x over previous
import os

import jax
import jax.numpy as jnp
from jax import lax
from jax.experimental import pallas as pl
from jax.experimental.pallas import tpu as pltpu

os.makedirs("/tmp/jax_kernel_cache", exist_ok=True)
jax.config.update("jax_compilation_cache_dir", "/tmp/jax_kernel_cache")
jax.config.update("jax_persistent_cache_min_compile_time_secs", 0)
jax.config.update("jax_persistent_cache_min_entry_size_bytes", -1)

N_DEV = 8
B = 2
S = 256
HQ = 4
DH = 64
D_MODEL = 512
D_QK = 256
NB = 4
BLK = 64
SW = N_DEV * BLK


def kernel(x, Wq, K_ext, V_ext, Wo):
    def body(x_ref, wq_ref, k_ref, v_ref, wo_ref, out_ref,
             stage, W, ctxsrc, ctxbuf,
             qkv_send, qkv_recv, ctx_send, ctx_recv):
        my = lax.axis_index("i")

        barrier_sem = pltpu.get_barrier_semaphore()
        for d in range(1, N_DEV):
            pl.semaphore_signal(
                barrier_sem, inc=1,
                device_id=(my ^ d,), device_id_type=pl.DeviceIdType.MESH,
            )
        pl.semaphore_wait(barrier_sem, N_DEV - 1)

        wq_bf = wq_ref[...].astype(jnp.bfloat16)
        for b in range(B):
            qb = jnp.dot(x_ref[b].astype(jnp.bfloat16), wq_bf,
                         preferred_element_type=jnp.float32
                         ).astype(jnp.bfloat16)
            for r in range(NB):
                d = b * NB + r
                rows = pl.ds(r * BLK, BLK)
                stage[d, 0] = qb[r * BLK:(r + 1) * BLK, :]
                stage[d, 1] = k_ref[b, rows, :, :].reshape(
                    BLK, D_QK).astype(jnp.bfloat16)
                stage[d, 2] = v_ref[b, rows, :, :].reshape(
                    BLK, D_QK).astype(jnp.bfloat16)

        qkv_dmas = []
        for d in range(N_DEV):
            dma = pltpu.make_async_remote_copy(
                src_ref=stage.at[d], dst_ref=W.at[my],
                send_sem=qkv_send.at[d], recv_sem=qkv_recv,
                device_id=(d,), device_id_type=pl.DeviceIdType.MESH,
            )
            qkv_dmas.append(dma)

        @pl.when(my != 0)
        def _():
            qkv_dmas[0].start()

        @pl.when(my == 0)
        def _():
            W[pl.ds(0, 1)] = stage[pl.ds(0, 1)]

        for d in range(1, N_DEV):
            @pl.when(my != d)
            def _(d=d):
                qkv_dmas[d].start()

            @pl.when(my == d)
            def _(d=d):
                W[pl.ds(d, 1)] = stage[pl.ds(d, 1)]

        for j in range(N_DEV - 1):
            pltpu.make_async_remote_copy(
                src_ref=stage.at[j], dst_ref=W.at[j],
                send_sem=qkv_send.at[j], recv_sem=qkv_recv,
                device_id=(my,), device_id_type=pl.DeviceIdType.MESH,
            ).wait_recv()

        qcat = W[:, 0, :, :].reshape(SW, D_QK)
        kcat = W[:, 1, :, :].reshape(SW, D_QK)
        vcat = W[:, 2, :, :].reshape(SW, D_QK)
        ctx_heads = []
        for h in range(HQ):
            cols = slice(h * DH, (h + 1) * DH)
            scores = lax.dot_general(
                qcat[:, cols], kcat[:, cols], (((1,), (1,)), ((), ())),
                preferred_element_type=jnp.float32,
            ) * 0.125
            m = jnp.max(scores, axis=1, keepdims=True)
            w = jnp.exp(scores - m)
            w = (w / jnp.sum(w, axis=1, keepdims=True)).astype(jnp.bfloat16)
            ctx_heads.append(
                jnp.dot(w, vcat[:, cols], preferred_element_type=jnp.float32))
        ctx = jnp.concatenate(ctx_heads, axis=1)
        ctxsrc[...] = ctx.reshape(N_DEV, BLK, D_QK).astype(jnp.bfloat16)

        ctx_dmas = []
        for s in range(N_DEV):
            dma = pltpu.make_async_remote_copy(
                src_ref=ctxsrc.at[s], dst_ref=ctxbuf.at[my],
                send_sem=ctx_send.at[s], recv_sem=ctx_recv,
                device_id=(s,), device_id_type=pl.DeviceIdType.MESH,
            )
            ctx_dmas.append(dma)
            @pl.when(my != s)
            def _(s=s):
                dma.start()

        @pl.when(my == 0)
        def _():
            ctxbuf[pl.ds(0, 1)] = ctxsrc[pl.ds(0, 1)]
        for s in range(1, N_DEV):
            @pl.when(my == s)
            def _(s=s):
                ctxbuf[pl.ds(s, 1)] = ctxsrc[pl.ds(s, 1)]

        for j in range(N_DEV - 1):
            pltpu.make_async_remote_copy(
                src_ref=ctxsrc.at[j], dst_ref=ctxbuf.at[j],
                send_sem=ctx_send.at[j], recv_sem=ctx_recv,
                device_id=(my,), device_id_type=pl.DeviceIdType.MESH,
            ).wait_recv()

        wo_bf = wo_ref[...].astype(jnp.bfloat16)
        for b in range(B):
            ctx_b = jnp.concatenate(
                [ctxbuf[b * NB + r] for r in range(NB)], axis=0)
            out_ref[b] = jnp.dot(ctx_b, wo_bf,
                                 preferred_element_type=jnp.float32)

        for d in range(N_DEV):
            @pl.when(my != d)
            def _(d=d):
                qkv_dmas[d].wait_send()
                ctx_dmas[d].wait_send()

    return pl.pallas_call(
        body,
        out_shape=jax.ShapeDtypeStruct((B, S, D_MODEL), jnp.float32),
        in_specs=[pl.BlockSpec(memory_space=pltpu.VMEM)] * 5,
        out_specs=pl.BlockSpec(memory_space=pltpu.VMEM),
        scratch_shapes=[
            pltpu.VMEM((N_DEV, 3, BLK, D_QK), jnp.bfloat16),
            pltpu.VMEM((N_DEV, 3, BLK, D_QK), jnp.bfloat16),
            pltpu.VMEM((N_DEV, BLK, D_QK), jnp.bfloat16),
            pltpu.VMEM((N_DEV, BLK, D_QK), jnp.bfloat16),
            pltpu.SemaphoreType.DMA((N_DEV,)),
            pltpu.SemaphoreType.DMA,
            pltpu.SemaphoreType.DMA((N_DEV,)),
            pltpu.SemaphoreType.DMA,
        ],
        compiler_params=pltpu.CompilerParams(collective_id=0),
    )(x, Wq, K_ext, V_ext, Wo)


# device time: 23106 ns/iter; 8.4284x vs baseline; 1.0277x over previous
import os

import jax
import jax.numpy as jnp
from jax import lax
from jax.experimental import pallas as pl
from jax.experimental.pallas import tpu as pltpu

os.makedirs("/tmp/jax_kernel_cache", exist_ok=True)
jax.config.update("jax_compilation_cache_dir", "/tmp/jax_kernel_cache")
jax.config.update("jax_persistent_cache_min_compile_time_secs", 0)
jax.config.update("jax_persistent_cache_min_entry_size_bytes", -1)

N_DEV = 8
B = 2
S = 256
HQ = 4
DH = 64
D_MODEL = 512
D_QK = 256
NB = 4
BLK = 64
SW = N_DEV * BLK


def kernel(x, Wq, K_ext, V_ext, Wo):
    def body(x_ref, wq_ref, k_ref, v_ref, wo_ref, out_ref,
             stage, W, ctxsrc, ctxbuf,
             qkv_send, qkv_recv, ctx_send, ctx_recv):
        my = lax.axis_index("i")

        barrier_sem = pltpu.get_barrier_semaphore()
        for d in range(1, N_DEV):
            pl.semaphore_signal(
                barrier_sem, inc=1,
                device_id=(my ^ d,), device_id_type=pl.DeviceIdType.MESH,
            )

        wq_bf = wq_ref[...].astype(jnp.bfloat16)
        for b in range(B):
            qb = jnp.dot(x_ref[b].astype(jnp.bfloat16), wq_bf,
                         preferred_element_type=jnp.float32
                         ).astype(jnp.bfloat16)
            for r in range(NB):
                d = b * NB + r
                rows = pl.ds(r * BLK, BLK)
                stage[d, 0] = qb[r * BLK:(r + 1) * BLK, :]
                stage[d, 1] = k_ref[b, rows, :, :].reshape(
                    BLK, D_QK).astype(jnp.bfloat16)
                stage[d, 2] = v_ref[b, rows, :, :].reshape(
                    BLK, D_QK).astype(jnp.bfloat16)

        pl.semaphore_wait(barrier_sem, N_DEV - 1)

        qkv_dmas = []
        for d in range(N_DEV):
            dma = pltpu.make_async_remote_copy(
                src_ref=stage.at[d], dst_ref=W.at[my],
                send_sem=qkv_send.at[d], recv_sem=qkv_recv,
                device_id=(d,), device_id_type=pl.DeviceIdType.MESH,
            )
            qkv_dmas.append(dma)

        @pl.when(my != 0)
        def _():
            qkv_dmas[0].start()

        @pl.when(my == 0)
        def _():
            W[pl.ds(0, 1)] = stage[pl.ds(0, 1)]

        for d in range(1, N_DEV):
            @pl.when(my != d)
            def _(d=d):
                qkv_dmas[d].start()

            @pl.when(my == d)
            def _(d=d):
                W[pl.ds(d, 1)] = stage[pl.ds(d, 1)]

        for j in range(N_DEV - 1):
            pltpu.make_async_remote_copy(
                src_ref=stage.at[j], dst_ref=W.at[j],
                send_sem=qkv_send.at[j], recv_sem=qkv_recv,
                device_id=(my,), device_id_type=pl.DeviceIdType.MESH,
            ).wait_recv()

        qcat = W[:, 0, :, :].reshape(SW, D_QK)
        kcat = W[:, 1, :, :].reshape(SW, D_QK)
        vcat = W[:, 2, :, :].reshape(SW, D_QK)
        ctx_heads = []
        for h in range(HQ):
            cols = slice(h * DH, (h + 1) * DH)
            scores = lax.dot_general(
                qcat[:, cols], kcat[:, cols], (((1,), (1,)), ((), ())),
                preferred_element_type=jnp.float32,
            ) * 0.125
            m = jnp.max(scores, axis=1, keepdims=True)
            w = jnp.exp(scores - m)
            w = (w / jnp.sum(w, axis=1, keepdims=True)).astype(jnp.bfloat16)
            ctx_heads.append(
                jnp.dot(w, vcat[:, cols], preferred_element_type=jnp.float32))
        ctx = jnp.concatenate(ctx_heads, axis=1)
        ctxsrc[...] = ctx.reshape(N_DEV, BLK, D_QK).astype(jnp.bfloat16)

        ctx_dmas = []
        for s in range(N_DEV):
            dma = pltpu.make_async_remote_copy(
                src_ref=ctxsrc.at[s], dst_ref=ctxbuf.at[my],
                send_sem=ctx_send.at[s], recv_sem=ctx_recv,
                device_id=(s,), device_id_type=pl.DeviceIdType.MESH,
            )
            ctx_dmas.append(dma)
            @pl.when(my != s)
            def _(s=s):
                dma.start()

        @pl.when(my == 0)
        def _():
            ctxbuf[pl.ds(0, 1)] = ctxsrc[pl.ds(0, 1)]
        for s in range(1, N_DEV):
            @pl.when(my == s)
            def _(s=s):
                ctxbuf[pl.ds(s, 1)] = ctxsrc[pl.ds(s, 1)]

        wo_bf = wo_ref[...].astype(jnp.bfloat16)

        for j in range(N_DEV - 1):
            pltpu.make_async_remote_copy(
                src_ref=ctxsrc.at[j], dst_ref=ctxbuf.at[j],
                send_sem=ctx_send.at[j], recv_sem=ctx_recv,
                device_id=(my,), device_id_type=pl.DeviceIdType.MESH,
            ).wait_recv()

        for b in range(B):
            ctx_b = jnp.concatenate(
                [ctxbuf[b * NB + r] for r in range(NB)], axis=0)
            out_ref[b] = jnp.dot(ctx_b, wo_bf,
                                 preferred_element_type=jnp.float32)

        for d in range(N_DEV):
            @pl.when(my != d)
            def _(d=d):
                qkv_dmas[d].wait_send()
                ctx_dmas[d].wait_send()

    return pl.pallas_call(
        body,
        out_shape=jax.ShapeDtypeStruct((B, S, D_MODEL), jnp.float32),
        in_specs=[pl.BlockSpec(memory_space=pltpu.VMEM)] * 5,
        out_specs=pl.BlockSpec(memory_space=pltpu.VMEM),
        scratch_shapes=[
            pltpu.VMEM((N_DEV, 3, BLK, D_QK), jnp.bfloat16),
            pltpu.VMEM((N_DEV, 3, BLK, D_QK), jnp.bfloat16),
            pltpu.VMEM((N_DEV, BLK, D_QK), jnp.bfloat16),
            pltpu.VMEM((N_DEV, BLK, D_QK), jnp.bfloat16),
            pltpu.SemaphoreType.DMA((N_DEV,)),
            pltpu.SemaphoreType.DMA,
            pltpu.SemaphoreType.DMA((N_DEV,)),
            pltpu.SemaphoreType.DMA,
        ],
        compiler_params=pltpu.CompilerParams(collective_id=0),
    )(x, Wq, K_ext, V_ext, Wo)
